# baseline (speedup 1.0000x reference)
import jax
import jax.numpy as jnp
from jax import lax
from jax.experimental import pallas as pl
from jax.experimental.pallas import tpu as pltpu
from jax.experimental.pallas import tpu_sc as plsc

B = 16384
D = 16
NW = 32
BPW = B // NW
WAVE = 8
NWAVE = BPW // WAVE
DEPTH = 3

NROWS = 1000000


def _mf_body(user_hbm, item_hbm, uft_hbm, ift_hbm, out_hbm,
             uidx_v, vidx_v, ubuf_v, vbuf_v, uscr_v, vscr_v, out_v,
             *sems):
    sem_u = sems[:DEPTH]
    sem_v = sems[DEPTH:]
    c = lax.axis_index("c")
    s = lax.axis_index("s")
    wid = s * 2 + c
    base = wid * BPW

    pltpu.sync_copy(user_hbm.at[pl.ds(base, BPW)], uidx_v.at[pl.ds(0, BPW)])
    pltpu.sync_copy(item_hbm.at[pl.ds(base, BPW)], vidx_v.at[pl.ds(0, BPW)])

    lane = lax.iota(jnp.int32, 16)

    def fire(w, slot):
        u = uidx_v[pl.ds(w * WAVE, 16)]
        v = vidx_v[pl.ds(w * WAVE, 16)]
        for e in range(WAVE):
            uoff = pl.multiple_of((u[e] >> 7) * 128, 128)
            voff = pl.multiple_of((v[e] >> 7) * 128, 128)
            dsl = pl.ds(slot * (WAVE * 128) + e * 128, 128)
            pltpu.async_copy(uft_hbm.at[:, pl.ds(uoff, 128)],
                             ubuf_v.at[:, dsl], sem_u[slot])
            pltpu.async_copy(ift_hbm.at[:, pl.ds(voff, 128)],
                             vbuf_v.at[:, dsl], sem_v[slot])

    def drain_extract(w, slot):
        dsl = pl.ds(0, WAVE * 128)
        pltpu.make_async_copy(uft_hbm.at[:, dsl], ubuf_v.at[:, dsl],
                              sem_u[slot]).wait()
        pltpu.make_async_copy(ift_hbm.at[:, dsl], vbuf_v.at[:, dsl],
                              sem_v[slot]).wait()
        u = uidx_v[pl.ds(w * WAVE, 16)]
        v = vidx_v[pl.ds(w * WAVE, 16)]
        half = w & 1
        for e in range(WAVE):
            boff = slot * (WAVE * 128) + e * 128
            ucol = jnp.full((16,), (u[e] & 127) + boff, jnp.int32)
            vcol = jnp.full((16,), (v[e] & 127) + boff, jnp.int32)
            ue = plsc.load_gather(ubuf_v, [lane, ucol])
            ve = plsc.load_gather(vbuf_v, [lane, vcol])
            ecol = jnp.full((16,), e, jnp.int32) + (half << 3)
            plsc.store_scatter(uscr_v, [lane, ecol], ue)
            plsc.store_scatter(vscr_v, [lane, ecol], ve)

        @pl.when(half == 1)
        def _():
            acc = jnp.zeros((16,), jnp.float32)
            for d in range(D):
                acc = acc + uscr_v[d] * vscr_v[d]
            out_v[pl.ds((w >> 1) * 16, 16)] = acc

    def body(w, _):
        wm = w - (w // DEPTH) * DEPTH
        om = (w - (DEPTH - 1))
        omm = om - (om // DEPTH) * DEPTH
        for slot in range(DEPTH):
            @pl.when(jnp.logical_and(wm == slot, w < NWAVE))
            def _(slot=slot):
                fire(w, slot)

            @pl.when(jnp.logical_and(omm == slot, w >= DEPTH - 1))
            def _(slot=slot):
                drain_extract(w - (DEPTH - 1), slot)
        return ()

    lax.fori_loop(0, NWAVE + DEPTH - 1, body, ())

    pltpu.sync_copy(out_v, out_hbm.at[pl.ds(base, BPW)])


def kernel(user, item, user_factors, item_factors):
    mesh = plsc.VectorSubcoreMesh(core_axis_name="c", subcore_axis_name="s")
    k = pl.kernel(
        _mf_body,
        out_type=jax.ShapeDtypeStruct((B,), jnp.float32),
        mesh=mesh,
        compiler_params=pltpu.CompilerParams(
            needs_layout_passes=False, use_tc_tiling_on_sc=True),
        scratch_types=[
            pltpu.VMEM((BPW + 16,), jnp.int32),
            pltpu.VMEM((BPW + 16,), jnp.int32),
            pltpu.VMEM((D, DEPTH * WAVE * 128), jnp.float32),
            pltpu.VMEM((D, DEPTH * WAVE * 128), jnp.float32),
            pltpu.VMEM((D, 16), jnp.float32),
            pltpu.VMEM((D, 16), jnp.float32),
            pltpu.VMEM((BPW,), jnp.float32),
        ] + [pltpu.SemaphoreType.DMA] * (2 * DEPTH),
    )
    return k(user, item, user_factors.T, item_factors.T)
